# baseline (speedup 1.0000x reference)
import numpy as np
import jax
import jax.numpy as jnp
from jax.experimental import pallas as pl
from jax.experimental.pallas import tpu as pltpu

N_Q = 4
N_KER = 4
K, STRIDE, HOUT = 4, 2, 3
EPS = 1e-12
NPATCH = HOUT * HOUT
NFEAT = 3 * N_KER
NF = NPATCH * NFEAT
FPAD = 128
NU = 16
BBLK = 1024

_HP = jax.lax.Precision.HIGHEST


def _patch_tensor():
    G = np.zeros((64, NPATCH, NU), np.float32)
    for i in range(HOUT):
        for j in range(HOUT):
            q = i * HOUT + j
            for r in range(K):
                for c in range(K):
                    G[8 * (STRIDE * i + r) + STRIDE * j + c, q, r * K + c] = 1.0
    return G


def _cnot_ring():
    P = np.eye(16, dtype=np.float32)

    def cnot(c_, t_):
        M = np.zeros((16, 16), np.float32)
        for n in range(16):
            bits = [(n >> (3 - w)) & 1 for w in range(N_Q)]
            if bits[c_]:
                bits[t_] ^= 1
            m = sum(b << (3 - w) for w, b in enumerate(bits))
            M[m, n] = 1.0
        return M

    for (c_, t_) in [(0, 1), (1, 2), (2, 3), (3, 0)]:
        P = cnot(c_, t_) @ P
    return P


def _build_A(qweights):
    phi, theta, omega = qweights[..., 0], qweights[..., 1], qweights[..., 2]
    c, s = jnp.cos(theta / 2), jnp.sin(theta / 2)
    ep = jnp.exp(-0.5j * (phi + omega).astype(jnp.complex64))
    em = jnp.exp(0.5j * (phi - omega).astype(jnp.complex64))
    m00, m01, m10, m11 = ep * c, -em * s, jnp.conj(em) * s, jnp.conj(ep) * c
    R = jnp.stack([jnp.stack([m00, m01], -1),
                   jnp.stack([m10, m11], -1)], -2)

    P = jnp.asarray(_cnot_ring()).astype(jnp.complex64)
    X = np.array([[0, 1], [1, 0]], np.complex64)
    Y = np.array([[0, -1j], [1j, 0]], np.complex64)
    Z = np.array([[1, 0], [0, -1]], np.complex64)
    I8 = np.eye(8, dtype=np.complex64)
    obs = [jnp.asarray(np.kron(o, I8)) for o in (X, Y, Z)]

    mats = []
    for k in range(N_KER):
        U = R[k, 0]
        for q in range(1, N_Q):
            U = jnp.kron(U, R[k, q])
        U = jnp.matmul(P, U, precision=_HP)
        Uh = jnp.conj(U.T)
        for O in obs:
            M = jnp.matmul(Uh, jnp.matmul(O, U, precision=_HP), precision=_HP)
            mats.append(jnp.real(M))
    return jnp.stack(mats, 0).astype(jnp.float32)


def _g2_matrix():
    G3 = _patch_tensor()
    G2 = np.zeros((64, NU, NPATCH, NFEAT), np.float32)
    G2 += G3.transpose(0, 2, 1)[:, :, :, None]
    G2 = G2.reshape(64, NU, NF)
    out = np.zeros((64, NU, FPAD), np.float32)
    out[:, :, :NF] = G2
    return out.reshape(64, NU * FPAD)


def _gn_matrix():
    G3 = _patch_tensor()
    pm = G3.sum(2)
    Gn = np.repeat(pm, NFEAT, axis=1)
    out = np.zeros((64, FPAD), np.float32)
    out[:, :NF] = Gn
    return out


def _sel2_matrix():
    return np.tile(np.eye(FPAD, dtype=np.float32), (NU, 1))


def _perm():
    idx = np.zeros(NF, np.int32)
    for q in range(NPATCH):
        for j in range(NFEAT):
            idx[q * NFEAT + j] = j * NPATCH + q
    return idx


def _leaky(v):
    return jnp.where(v >= 0, v, 0.1 * v)


def _body(x_ref, c_ref, g2_ref, gn_ref, s_ref, w1_ref, b1_ref, w2_ref,
          b2_ref, o_ref):
    xv = x_ref[...]
    T2 = jnp.dot(xv, c_ref[...], precision=_HP, preferred_element_type=jnp.float32)
    P2 = jnp.dot(xv, g2_ref[...], precision=_HP, preferred_element_type=jnp.float32)
    nsq = jnp.dot(xv * xv, gn_ref[...], precision=_HP, preferred_element_type=jnp.float32)
    E = jnp.dot(T2 * P2, s_ref[...], precision=_HP, preferred_element_type=jnp.float32)
    den = jnp.sqrt(nsq) + EPS
    feats = _leaky(E / (den * den))
    h1 = _leaky(jnp.dot(feats, w1_ref[...], precision=_HP,
                        preferred_element_type=jnp.float32) + b1_ref[...])
    o_ref[...] = jnp.dot(h1, w2_ref[...], precision=_HP,
                         preferred_element_type=jnp.float32) + b2_ref[...]


def kernel(x, qweights, fc1_w, fc1_b, fc2_w, fc2_b):
    Bsz = x.shape[0]
    xf = x.reshape(Bsz, 64)
    A = _build_A(qweights)
    G3 = jnp.asarray(_patch_tensor())
    C = jnp.einsum('xqv,juv->xuqj', G3, A, precision=_HP)
    C = jnp.concatenate(
        [C.reshape(64, NU, NF),
         jnp.zeros((64, NU, FPAD - NF), jnp.float32)], axis=2)
    C = C.reshape(64, NU * FPAD)
    G2 = jnp.asarray(_g2_matrix())
    Gn = jnp.asarray(_gn_matrix())
    S2 = jnp.asarray(_sel2_matrix())
    W1 = jnp.zeros((FPAD, 32), jnp.float32).at[:NF].set(
        fc1_w[:, jnp.asarray(_perm())].T)
    W2 = fc2_w.T
    b1 = fc1_b.reshape(1, 32)
    b2 = fc2_b.reshape(1, 3)

    def const(shape):
        return pl.BlockSpec(shape, lambda i: (0, 0))

    return pl.pallas_call(
        _body,
        grid=(Bsz // BBLK,),
        in_specs=[
            pl.BlockSpec((BBLK, 64), lambda i: (i, 0)),
            const((64, NU * FPAD)),
            const((64, NU * FPAD)),
            const((64, FPAD)),
            const((NU * FPAD, FPAD)),
            const((FPAD, 32)),
            const((1, 32)),
            const((32, 3)),
            const((1, 3)),
        ],
        out_specs=pl.BlockSpec((BBLK, 3), lambda i: (i, 0)),
        out_shape=jax.ShapeDtypeStruct((Bsz, 3), jnp.float32),
        compiler_params=pltpu.CompilerParams(
            dimension_semantics=("parallel",),
        ),
        name="qccnn_fused_v2",
    )(xf, C, G2, Gn, S2, W1, b1, W2, b2)
